# baseline (speedup 1.0000x reference)
import jax
import jax.numpy as jnp
from jax.experimental import pallas as pl
from jax.experimental.pallas import tpu as pltpu

_EPS = 1e-06
_HD = 512
_WD = 512


def _acc_body(nbr_ref, cam_ref, d_ref, o_ref):
    i = pl.program_id(0)
    c = pl.program_id(1)

    c0 = cam_ref[0]
    c1 = cam_ref[1]

    def _pre(j):
        t = jnp.abs((c0 - nbr_ref[j, 0].astype(jnp.float32))
                    * (c1 - nbr_ref[j, 1].astype(jnp.float32)))
        return jnp.where(t <= _EPS, 0.0, t)

    pres = [_pre(j) for j in range(8)]
    s = pres[0]
    for j in range(1, 8):
        s = s + pres[j]
    flip = 7 - i
    w_pre = jnp.float32(0.0)
    for j in range(8):
        w_pre = jnp.where(flip == j, pres[j], w_pre)
    w = w_pre / s
    w = jnp.where(jnp.abs(w) <= _EPS, 0.0, w)

    n0 = nbr_ref[i, 0]
    n1 = nbr_ref[i, 1]
    rolled = pltpu.roll(d_ref[0], (_HD - n0) % _HD, axis=0)
    rolled = pltpu.roll(rolled, (_WD - n1) % _WD, axis=1)
    contrib = w * rolled

    @pl.when(i == 0)
    def _():
        o_ref[pl.ds(c, 1)] = contrib[None]

    @pl.when(i > 0)
    def _():
        o_ref[pl.ds(c, 1)] = o_ref[pl.ds(c, 1)] + contrib[None]


def _asm_body(t_ref, o_ref, scr_ref):
    c2 = pl.program_id(0)
    t = pl.program_id(1)

    @pl.when((c2 == 0) & (t == 0))
    def _():
        scr_ref[:, 0:_HD] = t_ref[...]
        scr_ref[:, _HD:2 * _HD] = t_ref[...]
        scr_ref[:, 2 * _HD:2 * _HD + 64] = t_ref[:, 0:64]

    q0 = 128 * t + c2
    j0 = q0 // 3
    r0 = c2 * 341 + j0
    off = q0 - 3 * j0
    r0a = (r0 // 8) * 8
    rem = r0 - r0a

    pts = []
    for k in range(3):
        pck = scr_ref[pl.ds(k, 1), pl.ds(r0a, 56), :][0]
        pck = pltpu.roll(pck, (56 - rem) % 56, axis=0)[0:48]
        pts.append(jnp.concatenate([pck, pck], axis=1))
    wch = jnp.stack(pts, axis=-1).reshape(48, 3072)
    ws = [wch[:, 1024 * ph:1024 * ph + 1024] for ph in range(3)]
    v = jnp.stack(ws, axis=1).reshape(144, 1024)
    v = pltpu.roll(v, (144 - off) % 144, axis=0)
    o_ref[0] = v[0:128]


def kernel(pixel, cam_xyz, neighbors, data):
    H, W = pixel.shape
    nbr = neighbors.astype(jnp.int32)
    camxy = cam_xyz[:2].astype(jnp.float32)
    dp = jnp.transpose(data.reshape(64, _HD, _WD, 3),
                       (0, 3, 1, 2)).reshape(192, _HD, _WD)

    acc_spec = pltpu.PrefetchScalarGridSpec(
        num_scalar_prefetch=2,
        grid=(8, 3),
        in_specs=[
            pl.BlockSpec(
                (1, _HD, _WD),
                lambda i, c, nref, cref: (
                    (nref[i, 0] * 8 + nref[i, 1]) * 3 + c, 0, 0),
            ),
        ],
        out_specs=pl.BlockSpec((3, _HD, _WD),
                               lambda i, c, nref, cref: (0, 0, 0)),
    )

    acc = pl.pallas_call(
        _acc_body,
        grid_spec=acc_spec,
        out_shape=jax.ShapeDtypeStruct((3, _HD, _WD), jnp.float32),
    )(nbr, camxy, dp)

    return pl.pallas_call(
        _asm_body,
        grid=(3, 8),
        in_specs=[pl.BlockSpec((3, _HD, _WD), lambda c2, t: (0, 0, 0))],
        out_specs=pl.BlockSpec((1, 128, 2 * _WD), lambda c2, t: (c2, t, 0)),
        out_shape=jax.ShapeDtypeStruct((3, 2 * _HD, 2 * _WD), jnp.float32),
        scratch_shapes=[pltpu.VMEM((3, 2 * _HD + 64, _WD), jnp.float32)],
    )(acc)
